# baseline (speedup 1.0000x reference)
import functools

import jax
import jax.numpy as jnp
import numpy as np
from jax import lax
from jax.experimental import pallas as pl
from jax.experimental.pallas import tpu as pltpu
from jax.experimental.pallas import tpu_sc as plsc


def _swizzle_matrix():
    cols = np.arange(128)
    k, r = cols // 32, cols % 32
    true_col = 32 * k + 16 * (r % 2) + r // 2
    m = np.zeros((128, 128), dtype=np.float32)
    m[true_col, cols] = 1.0
    return m


_PMAT = _swizzle_matrix()

N = 10000
E = 320000
D = 128
ED = 16
OUT = 16
G = 64
L = 3

BE = 4000
BN = 1000
BP = 1000


def _edge_embed_body(ea_ref, we_ref, be_ref, out_ref):
    out_ref[0] = (
        jnp.dot(ea_ref[...], we_ref[0], preferred_element_type=jnp.float32)
        + be_ref[0]
    ).astype(jnp.bfloat16)


def _edge_embed(edge_attr, We, be):
    grid = (L, E // BE)
    return pl.pallas_call(
        _edge_embed_body,
        grid=grid,
        in_specs=[
            pl.BlockSpec((BE, ED), lambda l, i: (i, 0)),
            pl.BlockSpec((1, ED, D), lambda l, i: (l, 0, 0)),
            pl.BlockSpec((1, 1, D), lambda l, i: (l, 0, 0)),
        ],
        out_specs=pl.BlockSpec((1, BE, D), lambda l, i: (l, i, 0)),
        out_shape=jax.ShapeDtypeStruct((L, E, D), jnp.bfloat16),
    )(edge_attr, We, be)


NC = 2
NS = 16
NW = NC * NS
EPW = E // NW
CH = 40
NCHUNK = EPW // CH
PADN = 10112
RPS = PADN // NS
MB = 5
GB = 2
DB = 10
UNROLL = 10


def _mp_body(l, h_hbm, e_hbm, src_hbm, dst_hbm, out_hbm,
             sbuf, dbuf, gbuf, ebuf, mbuf, agg_sh,
             sem_si, sem_di, sem_g, sem_e, sem_sc):
    c = lax.axis_index("c")
    s = lax.axis_index("s")
    wid = c * NS + s
    ebase = wid * EPW

    def e_src(j):
        return e_hbm.at[l, pl.ds(ebase + j * CH, CH)]

    def issue_sidx(j, b):
        return pltpu.async_copy(src_hbm.at[pl.ds(ebase + j * CH, CH)],
                                sbuf[b], sem_si.at[b])

    def issue_didx(j, b):
        return pltpu.async_copy(dst_hbm.at[pl.ds(ebase + j * CH, CH)],
                                dbuf[b], sem_di.at[b])

    def gather_desc(sb, gb):
        return pltpu.make_async_copy(h_hbm.at[sbuf[sb]], gbuf[gb],
                                     sem_g.at[gb])

    def scat_desc(mb, db):
        return pltpu.make_async_copy(mbuf[mb], agg_sh.at[dbuf[db]],
                                     sem_sc.at[mb])

    @plsc.parallel_loop(0, CH, unroll=4)
    def _zrow(r):
        for k in range(D // 16):
            mbuf[0][r, pl.ds(k * 16, 16)] = jnp.zeros((16,), jnp.float32)

    for i in range(RPS // CH):
        pltpu.sync_copy(mbuf[0], agg_sh.at[pl.ds(s * RPS + i * CH, CH)])
    rem = RPS % CH
    if rem:
        pltpu.sync_copy(mbuf[0].at[pl.ds(0, rem)],
                        agg_sh.at[pl.ds(s * RPS + RPS - rem, rem)])
    plsc.subcore_barrier()

    for b in range(MB):
        issue_sidx(b, b)
    for b in range(DB):
        issue_didx(b, b)
    for b in range(GB):
        pltpu.async_copy(e_src(b), ebuf[b], sem_e.at[b])
        pltpu.make_async_copy(src_hbm.at[pl.ds(ebase, CH)], sbuf[b],
                              sem_si.at[b]).wait()
        gather_desc(b, b).start()

    def _group(g, carry):
        for u in range(UNROLL):
            b5 = u % MB
            b2 = u % GB
            d10 = u % DB
            j = g * UNROLL + u
            pltpu.make_async_copy(h_hbm.at[sbuf[b5]], gbuf[b2],
                                  sem_g.at[b2]).wait()
            pltpu.make_async_copy(e_src(j), ebuf[b2], sem_e.at[b2]).wait()

            @pl.when(j >= MB)
            def _drain():
                scat_desc(b5, d10).wait()

            @pl.when(jnp.logical_and(j >= MB, j + MB < NCHUNK))
            def _pdi():
                issue_didx(j + MB, (d10 + MB) % DB)

            @plsc.parallel_loop(0, CH, unroll=2)
            def _row(r):
                for k in range(D // 32):
                    w = ebuf[b2][r, pl.ds(k * 16, 16)]
                    elo = lax.bitcast_convert_type(w << 16, jnp.float32)
                    ehi = lax.bitcast_convert_type(
                        w & jnp.int32(-65536), jnp.float32)
                    sl0 = pl.ds(k * 32, 16)
                    sl1 = pl.ds(k * 32 + 16, 16)
                    mbuf[b5][r, sl0] = jnp.maximum(
                        elo + gbuf[b2][r, sl0], 0.0)
                    mbuf[b5][r, sl1] = jnp.maximum(
                        ehi + gbuf[b2][r, sl1], 0.0)

            pltpu.make_async_copy(dst_hbm.at[pl.ds(ebase, CH)], dbuf[d10],
                                  sem_di.at[d10]).wait()
            scat_desc(b5, d10).start(add=True)

            @pl.when(j + MB < NCHUNK)
            def _psi():
                issue_sidx(j + MB, b5)

            @pl.when(j + GB < NCHUNK)
            def _pg():
                sb = (b5 + GB) % MB
                pltpu.make_async_copy(src_hbm.at[pl.ds(ebase, CH)],
                                      sbuf[sb], sem_si.at[sb]).wait()
                gather_desc(sb, b2).start()
                pltpu.async_copy(e_src(j + GB), ebuf[b2], sem_e.at[b2])
        return carry

    lax.fori_loop(0, NCHUNK // UNROLL, _group, 0)
    for b in range(MB):
        pltpu.make_async_copy(mbuf[b], agg_sh.at[dbuf[0]],
                              sem_sc.at[b]).wait()
    plsc.subcore_barrier()
    pltpu.sync_copy(agg_sh.at[pl.ds(s * RPS, RPS)],
                    out_hbm.at[c, pl.ds(s * RPS, RPS)])


def _mp_layer(h, e32, src, dst, l):
    body = functools.partial(_mp_body, l)
    return pl.kernel(
        body,
        out_type=jax.ShapeDtypeStruct((NC, PADN, D), jnp.float32),
        mesh=plsc.VectorSubcoreMesh(core_axis_name="c", subcore_axis_name="s",
                                    num_cores=NC, num_subcores=NS),
        scratch_types=[
            [pltpu.VMEM((CH,), jnp.int32)] * MB,
            [pltpu.VMEM((CH,), jnp.int32)] * DB,
            [pltpu.VMEM((CH, D), jnp.float32)] * GB,
            [pltpu.VMEM((CH, D // 2), jnp.int32)] * GB,
            [pltpu.VMEM((CH, D), jnp.float32)] * MB,
            pltpu.VMEM_SHARED((PADN, D), jnp.float32),
            pltpu.SemaphoreType.DMA((MB,)),
            pltpu.SemaphoreType.DMA((DB,)),
            pltpu.SemaphoreType.DMA((GB,)),
            pltpu.SemaphoreType.DMA((GB,)),
            pltpu.SemaphoreType.DMA((MB,)),
        ],
    )(h, e32, src, dst)


def _mlp_body(h_ref, a0_ref, a1_ref, w1_ref, b1_ref, w2_ref, b2_ref, out_ref):
    z = h_ref[...] + a0_ref[0] + a1_ref[0]
    u = jnp.maximum(jnp.dot(z, w1_ref[...], preferred_element_type=jnp.float32)
                    + b1_ref[...], 0.0)
    v = jnp.dot(u, w2_ref[...], preferred_element_type=jnp.float32) + b2_ref[...]
    out_ref[...] = jnp.maximum(v, 0.0)


def _mlp(h, agg2, W1l, b1l, W2l, b2l):
    grid = (N // BN,)
    return pl.pallas_call(
        _mlp_body,
        grid=grid,
        in_specs=[
            pl.BlockSpec((BN, D), lambda i: (i, 0)),
            pl.BlockSpec((1, BN, D), lambda i: (0, i, 0)),
            pl.BlockSpec((1, BN, D), lambda i: (1, i, 0)),
            pl.BlockSpec((D, D), lambda i: (0, 0)),
            pl.BlockSpec((1, D), lambda i: (0, 0)),
            pl.BlockSpec((D, D), lambda i: (0, 0)),
            pl.BlockSpec((1, D), lambda i: (0, 0)),
        ],
        out_specs=pl.BlockSpec((BN, D), lambda i: (i, 0)),
        out_shape=jax.ShapeDtypeStruct((N, D), jnp.float32),
    )(h, agg2, agg2, W1l, b1l, W2l, b2l)


def _pool_head_body(h_ref, batch_ref, wf1_ref, bf1_ref, wf2_ref, bf2_ref,
                    out_ref, acc_ref, cnt_ref):
    i = pl.program_id(0)

    @pl.when(i == 0)
    def _init():
        acc_ref[...] = jnp.zeros_like(acc_ref)
        cnt_ref[...] = jnp.zeros_like(cnt_ref)

    seg = batch_ref[0, 0]
    gids = jax.lax.broadcasted_iota(jnp.int32, (G, BP), 0)
    onehot = (gids == seg[None, :]).astype(jnp.float32)
    acc_ref[...] += jnp.dot(onehot, h_ref[...],
                            preferred_element_type=jnp.float32)
    cnt_ref[...] += jnp.sum(onehot, axis=1, keepdims=True)

    @pl.when(i == pl.num_programs(0) - 1)
    def _fin():
        pooled = acc_ref[...] / jnp.maximum(cnt_ref[...], 1.0)
        hid = jnp.maximum(
            jnp.dot(pooled, wf1_ref[...], preferred_element_type=jnp.float32)
            + bf1_ref[...], 0.0)
        out_ref[...] = (jnp.dot(hid, wf2_ref[...],
                                preferred_element_type=jnp.float32)
                        + bf2_ref[...])


def _pool_head(h, batch, Wf1, bf1, Wf2, bf2):
    batch3 = batch.reshape(N // BP, 1, BP)
    grid = (N // BP,)
    return pl.pallas_call(
        _pool_head_body,
        grid=grid,
        in_specs=[
            pl.BlockSpec((BP, D), lambda i: (i, 0)),
            pl.BlockSpec((1, 1, BP), lambda i: (i, 0, 0)),
            pl.BlockSpec((D, D), lambda i: (0, 0)),
            pl.BlockSpec((1, D), lambda i: (0, 0)),
            pl.BlockSpec((D, OUT), lambda i: (0, 0)),
            pl.BlockSpec((1, OUT), lambda i: (0, 0)),
        ],
        out_specs=pl.BlockSpec((G, OUT), lambda i: (0, 0)),
        out_shape=jax.ShapeDtypeStruct((G, OUT), jnp.float32),
        scratch_shapes=[
            pltpu.VMEM((G, D), jnp.float32),
            pltpu.VMEM((G, 1), jnp.float32),
        ],
    )(h, batch3, Wf1, bf1, Wf2, bf2)


def kernel(x, edge_index, edge_attr, batch, We, be, W1, b1, W2, b2,
           Wf1, bf1, Wf2, bf2):
    src = edge_index[0]
    dst = edge_index[1]
    pmat = jnp.asarray(_PMAT)
    We_swz = We @ pmat
    be_swz = be @ pmat
    e_all = _edge_embed(edge_attr, We_swz, be_swz.reshape(L, 1, D))
    e32 = lax.bitcast_convert_type(
        e_all.reshape(L, E, D // 2, 2), jnp.int32)
    h = x
    for l in range(L):
        agg2 = _mp_layer(h, e32, src, dst, l)
        h = _mlp(h, agg2, W1[l], b1[l].reshape(1, D),
                 W2[l], b2[l].reshape(1, D))
    return _pool_head(h, batch, Wf1, bf1.reshape(1, D), Wf2, bf2.reshape(1, OUT))


# speedup vs baseline: 3.0952x; 3.0952x over previous
import functools

import jax
import jax.numpy as jnp
import numpy as np
from jax import lax
from jax.experimental import pallas as pl
from jax.experimental.pallas import tpu as pltpu
from jax.experimental.pallas import tpu_sc as plsc


def _half_selectors():
    w = np.arange(64)
    k, i = w // 16, w % 16
    s1 = np.zeros((128, 64), dtype=np.float32)
    s2 = np.zeros((128, 64), dtype=np.float32)
    s1[32 * k + i, w] = 1.0
    s2[32 * k + 16 + i, w] = 1.0
    return s1, s2


_S1, _S2 = _half_selectors()


def _bf16_bits_rne(x):
    u = lax.bitcast_convert_type(x, jnp.uint32)
    r = ((u >> 16) & 1) + jnp.uint32(0x7FFF)
    return (u + r) >> 16

N = 10000
E = 320000
D = 128
ED = 16
OUT = 16
G = 64
L = 3

BE = 4000
BN = 1000
BP = 1000


def _edge_embed_body(ea_ref, we_ref, be_ref, s1_ref, s2_ref, out_ref):
    t = (jnp.dot(ea_ref[...], we_ref[0], preferred_element_type=jnp.float32)
         + be_ref[0])
    a = jnp.dot(t, s1_ref[...], preferred_element_type=jnp.float32)
    b = jnp.dot(t, s2_ref[...], preferred_element_type=jnp.float32)
    packed = _bf16_bits_rne(a) | (_bf16_bits_rne(b) << 16)
    out_ref[0] = lax.bitcast_convert_type(packed, jnp.int32)


def _edge_embed(edge_attr, We, be, s1, s2):
    grid = (L, E // BE)
    return pl.pallas_call(
        _edge_embed_body,
        grid=grid,
        in_specs=[
            pl.BlockSpec((BE, ED), lambda l, i: (i, 0)),
            pl.BlockSpec((1, ED, D), lambda l, i: (l, 0, 0)),
            pl.BlockSpec((1, 1, D), lambda l, i: (l, 0, 0)),
            pl.BlockSpec((D, D // 2), lambda l, i: (0, 0)),
            pl.BlockSpec((D, D // 2), lambda l, i: (0, 0)),
        ],
        out_specs=pl.BlockSpec((1, BE, D // 2), lambda l, i: (l, i, 0)),
        out_shape=jax.ShapeDtypeStruct((L, E, D // 2), jnp.int32),
    )(edge_attr, We, be, s1, s2)


NC = 2
NS = 16
NW = NC * NS
EPW = E // NW
CH = 40
NCHUNK = EPW // CH
PADN = 10112
RPS = PADN // NS
MB = 5
GB = 2
DB = 10
UNROLL = 10


def _mp_body(l, h_hbm, e_hbm, src_hbm, dst_hbm, out_hbm,
             sbuf, dbuf, gbuf, ebuf, mbuf, agg_sh,
             sem_si, sem_di, sem_g, sem_e, sem_sc):
    c = lax.axis_index("c")
    s = lax.axis_index("s")
    wid = c * NS + s
    ebase = wid * EPW

    def e_src(j):
        return e_hbm.at[l, pl.ds(ebase + j * CH, CH)]

    def issue_sidx(j, b):
        return pltpu.async_copy(src_hbm.at[pl.ds(ebase + j * CH, CH)],
                                sbuf[b], sem_si.at[b])

    def issue_didx(j, b):
        return pltpu.async_copy(dst_hbm.at[pl.ds(ebase + j * CH, CH)],
                                dbuf[b], sem_di.at[b])

    def gather_desc(sb, gb):
        return pltpu.make_async_copy(h_hbm.at[sbuf[sb]], gbuf[gb],
                                     sem_g.at[gb])

    def scat_desc(mb, db):
        return pltpu.make_async_copy(mbuf[mb], agg_sh.at[dbuf[db]],
                                     sem_sc.at[mb])

    @plsc.parallel_loop(0, CH, unroll=4)
    def _zrow(r):
        for k in range(D // 16):
            mbuf[0][r, pl.ds(k * 16, 16)] = jnp.zeros((16,), jnp.float32)

    for i in range(RPS // CH):
        pltpu.sync_copy(mbuf[0], agg_sh.at[pl.ds(s * RPS + i * CH, CH)])
    rem = RPS % CH
    if rem:
        pltpu.sync_copy(mbuf[0].at[pl.ds(0, rem)],
                        agg_sh.at[pl.ds(s * RPS + RPS - rem, rem)])
    plsc.subcore_barrier()

    for b in range(MB):
        issue_sidx(b, b)
    for b in range(DB):
        issue_didx(b, b)
    for b in range(GB):
        pltpu.async_copy(e_src(b), ebuf[b], sem_e.at[b])
        pltpu.make_async_copy(src_hbm.at[pl.ds(ebase, CH)], sbuf[b],
                              sem_si.at[b]).wait()
        gather_desc(b, b).start()

    def _group(g, carry):
        for u in range(UNROLL):
            b5 = u % MB
            b2 = u % GB
            d10 = u % DB
            j = g * UNROLL + u
            pltpu.make_async_copy(h_hbm.at[sbuf[b5]], gbuf[b2],
                                  sem_g.at[b2]).wait()
            pltpu.make_async_copy(e_src(j), ebuf[b2], sem_e.at[b2]).wait()

            @pl.when(j >= MB)
            def _drain():
                scat_desc(b5, d10).wait()

            @pl.when(jnp.logical_and(j >= MB, j + MB < NCHUNK))
            def _pdi():
                issue_didx(j + MB, (d10 + MB) % DB)

            @plsc.parallel_loop(0, CH, unroll=2)
            def _row(r):
                for k in range(D // 32):
                    w = ebuf[b2][r, pl.ds(k * 16, 16)]
                    elo = lax.bitcast_convert_type(w << 16, jnp.float32)
                    ehi = lax.bitcast_convert_type(
                        w & jnp.int32(-65536), jnp.float32)
                    sl0 = pl.ds(k * 32, 16)
                    sl1 = pl.ds(k * 32 + 16, 16)
                    mbuf[b5][r, sl0] = jnp.maximum(
                        elo + gbuf[b2][r, sl0], 0.0)
                    mbuf[b5][r, sl1] = jnp.maximum(
                        ehi + gbuf[b2][r, sl1], 0.0)

            pltpu.make_async_copy(dst_hbm.at[pl.ds(ebase, CH)], dbuf[d10],
                                  sem_di.at[d10]).wait()
            scat_desc(b5, d10).start(add=True)

            @pl.when(j + MB < NCHUNK)
            def _psi():
                issue_sidx(j + MB, b5)

            @pl.when(j + GB < NCHUNK)
            def _pg():
                sb = (b5 + GB) % MB
                pltpu.make_async_copy(src_hbm.at[pl.ds(ebase, CH)],
                                      sbuf[sb], sem_si.at[sb]).wait()
                gather_desc(sb, b2).start()
                pltpu.async_copy(e_src(j + GB), ebuf[b2], sem_e.at[b2])
        return carry

    lax.fori_loop(0, NCHUNK // UNROLL, _group, 0)
    for b in range(MB):
        pltpu.make_async_copy(mbuf[b], agg_sh.at[dbuf[0]],
                              sem_sc.at[b]).wait()
    plsc.subcore_barrier()
    pltpu.sync_copy(agg_sh.at[pl.ds(s * RPS, RPS)],
                    out_hbm.at[c, pl.ds(s * RPS, RPS)])


def _mp_layer(h, e32, src, dst, l):
    body = functools.partial(_mp_body, l)
    return pl.kernel(
        body,
        out_type=jax.ShapeDtypeStruct((NC, PADN, D), jnp.float32),
        mesh=plsc.VectorSubcoreMesh(core_axis_name="c", subcore_axis_name="s",
                                    num_cores=NC, num_subcores=NS),
        scratch_types=[
            [pltpu.VMEM((CH,), jnp.int32)] * MB,
            [pltpu.VMEM((CH,), jnp.int32)] * DB,
            [pltpu.VMEM((CH, D), jnp.float32)] * GB,
            [pltpu.VMEM((CH, D // 2), jnp.int32)] * GB,
            [pltpu.VMEM((CH, D), jnp.float32)] * MB,
            pltpu.VMEM_SHARED((PADN, D), jnp.float32),
            pltpu.SemaphoreType.DMA((MB,)),
            pltpu.SemaphoreType.DMA((DB,)),
            pltpu.SemaphoreType.DMA((GB,)),
            pltpu.SemaphoreType.DMA((GB,)),
            pltpu.SemaphoreType.DMA((MB,)),
        ],
    )(h, e32, src, dst)


def _mlp_body(h_ref, a0_ref, a1_ref, w1_ref, b1_ref, w2_ref, b2_ref, out_ref):
    z = h_ref[...] + a0_ref[0] + a1_ref[0]
    u = jnp.maximum(jnp.dot(z, w1_ref[...], preferred_element_type=jnp.float32)
                    + b1_ref[...], 0.0)
    v = jnp.dot(u, w2_ref[...], preferred_element_type=jnp.float32) + b2_ref[...]
    out_ref[...] = jnp.maximum(v, 0.0)


def _mlp(h, agg2, W1l, b1l, W2l, b2l):
    grid = (N // BN,)
    return pl.pallas_call(
        _mlp_body,
        grid=grid,
        in_specs=[
            pl.BlockSpec((BN, D), lambda i: (i, 0)),
            pl.BlockSpec((1, BN, D), lambda i: (0, i, 0)),
            pl.BlockSpec((1, BN, D), lambda i: (1, i, 0)),
            pl.BlockSpec((D, D), lambda i: (0, 0)),
            pl.BlockSpec((1, D), lambda i: (0, 0)),
            pl.BlockSpec((D, D), lambda i: (0, 0)),
            pl.BlockSpec((1, D), lambda i: (0, 0)),
        ],
        out_specs=pl.BlockSpec((BN, D), lambda i: (i, 0)),
        out_shape=jax.ShapeDtypeStruct((N, D), jnp.float32),
    )(h, agg2, agg2, W1l, b1l, W2l, b2l)


def _pool_head_body(h_ref, batch_ref, wf1_ref, bf1_ref, wf2_ref, bf2_ref,
                    out_ref, acc_ref, cnt_ref):
    i = pl.program_id(0)

    @pl.when(i == 0)
    def _init():
        acc_ref[...] = jnp.zeros_like(acc_ref)
        cnt_ref[...] = jnp.zeros_like(cnt_ref)

    seg = batch_ref[0, 0]
    gids = jax.lax.broadcasted_iota(jnp.int32, (G, BP), 0)
    onehot = (gids == seg[None, :]).astype(jnp.float32)
    acc_ref[...] += jnp.dot(onehot, h_ref[...],
                            preferred_element_type=jnp.float32)
    cnt_ref[...] += jnp.sum(onehot, axis=1, keepdims=True)

    @pl.when(i == pl.num_programs(0) - 1)
    def _fin():
        pooled = acc_ref[...] / jnp.maximum(cnt_ref[...], 1.0)
        hid = jnp.maximum(
            jnp.dot(pooled, wf1_ref[...], preferred_element_type=jnp.float32)
            + bf1_ref[...], 0.0)
        out_ref[...] = (jnp.dot(hid, wf2_ref[...],
                                preferred_element_type=jnp.float32)
                        + bf2_ref[...])


def _pool_head(h, batch, Wf1, bf1, Wf2, bf2):
    batch3 = batch.reshape(N // BP, 1, BP)
    grid = (N // BP,)
    return pl.pallas_call(
        _pool_head_body,
        grid=grid,
        in_specs=[
            pl.BlockSpec((BP, D), lambda i: (i, 0)),
            pl.BlockSpec((1, 1, BP), lambda i: (i, 0, 0)),
            pl.BlockSpec((D, D), lambda i: (0, 0)),
            pl.BlockSpec((1, D), lambda i: (0, 0)),
            pl.BlockSpec((D, OUT), lambda i: (0, 0)),
            pl.BlockSpec((1, OUT), lambda i: (0, 0)),
        ],
        out_specs=pl.BlockSpec((G, OUT), lambda i: (0, 0)),
        out_shape=jax.ShapeDtypeStruct((G, OUT), jnp.float32),
        scratch_shapes=[
            pltpu.VMEM((G, D), jnp.float32),
            pltpu.VMEM((G, 1), jnp.float32),
        ],
    )(h, batch3, Wf1, bf1, Wf2, bf2)


def kernel(x, edge_index, edge_attr, batch, We, be, W1, b1, W2, b2,
           Wf1, bf1, Wf2, bf2):
    src = edge_index[0]
    dst = edge_index[1]
    e32 = _edge_embed(edge_attr, We, be.reshape(L, 1, D),
                      jnp.asarray(_S1), jnp.asarray(_S2))
    h = x
    for l in range(L):
        agg2 = _mp_layer(h, e32, src, dst, l)
        h = _mlp(h, agg2, W1[l], b1[l].reshape(1, D),
                 W2[l], b2[l].reshape(1, D))
    return _pool_head(h, batch, Wf1, bf1.reshape(1, D), Wf2, bf2.reshape(1, OUT))


# speedup vs baseline: 3.2274x; 1.0427x over previous
import functools

import jax
import jax.numpy as jnp
import numpy as np
from jax import lax
from jax.experimental import pallas as pl
from jax.experimental.pallas import tpu as pltpu
from jax.experimental.pallas import tpu_sc as plsc


def _bf16_bits_rne(x):
    u = lax.bitcast_convert_type(x, jnp.uint32)
    r = ((u >> 16) & 1) + jnp.uint32(0x7FFF)
    return (u + r) >> 16

N = 10000
E = 320000
D = 128
ED = 16
OUT = 16
G = 64
L = 3

BE = 4000
BN = 1000
BP = 1000


def _edge_embed_body(ea_ref, we_ref, be_ref, out_ref):
    t = (jnp.dot(ea_ref[...], we_ref[0], preferred_element_type=jnp.float32)
         + be_ref[0])
    packed = (_bf16_bits_rne(t[:, : D // 2])
              | (_bf16_bits_rne(t[:, D // 2:]) << 16))
    out_ref[0] = lax.bitcast_convert_type(packed, jnp.int32)


def _edge_embed(edge_attr, We, be):
    grid = (L, E // BE)
    return pl.pallas_call(
        _edge_embed_body,
        grid=grid,
        in_specs=[
            pl.BlockSpec((BE, ED), lambda l, i: (i, 0)),
            pl.BlockSpec((1, ED, D), lambda l, i: (l, 0, 0)),
            pl.BlockSpec((1, 1, D), lambda l, i: (l, 0, 0)),
        ],
        out_specs=pl.BlockSpec((1, BE, D // 2), lambda l, i: (l, i, 0)),
        out_shape=jax.ShapeDtypeStruct((L, E, D // 2), jnp.int32),
    )(edge_attr, We, be)


NC = 2
NS = 16
NW = NC * NS
EPW = E // NW
CH = 40
NCHUNK = EPW // CH
PADN = 10112
RPS = PADN // NS
MB = 5
GB = 2
DB = 10
UNROLL = 10


def _mp_body(l, h_hbm, e_hbm, src_hbm, dst_hbm, out_hbm,
             sbuf, dbuf, gbuf, ebuf, mbuf, agg_sh,
             sem_si, sem_di, sem_g, sem_e, sem_sc):
    c = lax.axis_index("c")
    s = lax.axis_index("s")
    wid = c * NS + s
    ebase = wid * EPW

    def e_src(j):
        return e_hbm.at[l, pl.ds(ebase + j * CH, CH)]

    def issue_sidx(j, b):
        return pltpu.async_copy(src_hbm.at[pl.ds(ebase + j * CH, CH)],
                                sbuf[b], sem_si.at[b])

    def issue_didx(j, b):
        return pltpu.async_copy(dst_hbm.at[pl.ds(ebase + j * CH, CH)],
                                dbuf[b], sem_di.at[b])

    def gather_desc(sb, gb):
        return pltpu.make_async_copy(h_hbm.at[sbuf[sb]], gbuf[gb],
                                     sem_g.at[gb])

    def scat_desc(mb, db):
        return pltpu.make_async_copy(mbuf[mb], agg_sh.at[dbuf[db]],
                                     sem_sc.at[mb])

    @plsc.parallel_loop(0, CH, unroll=4)
    def _zrow(r):
        for k in range(D // 16):
            mbuf[0][r, pl.ds(k * 16, 16)] = jnp.zeros((16,), jnp.float32)

    for i in range(RPS // CH):
        pltpu.sync_copy(mbuf[0], agg_sh.at[pl.ds(s * RPS + i * CH, CH)])
    rem = RPS % CH
    if rem:
        pltpu.sync_copy(mbuf[0].at[pl.ds(0, rem)],
                        agg_sh.at[pl.ds(s * RPS + RPS - rem, rem)])
    plsc.subcore_barrier()

    for b in range(MB):
        issue_sidx(b, b)
    for b in range(DB):
        issue_didx(b, b)
    for b in range(GB):
        pltpu.async_copy(e_src(b), ebuf[b], sem_e.at[b])
        pltpu.make_async_copy(src_hbm.at[pl.ds(ebase, CH)], sbuf[b],
                              sem_si.at[b]).wait()
        gather_desc(b, b).start()

    def _group(g, carry):
        for u in range(UNROLL):
            b5 = u % MB
            b2 = u % GB
            d10 = u % DB
            j = g * UNROLL + u
            pltpu.make_async_copy(h_hbm.at[sbuf[b5]], gbuf[b2],
                                  sem_g.at[b2]).wait()
            pltpu.make_async_copy(e_src(j), ebuf[b2], sem_e.at[b2]).wait()

            @pl.when(j >= MB)
            def _drain():
                scat_desc(b5, d10).wait()

            @pl.when(jnp.logical_and(j >= MB, j + MB < NCHUNK))
            def _pdi():
                issue_didx(j + MB, (d10 + MB) % DB)

            @plsc.parallel_loop(0, CH, unroll=2)
            def _row(r):
                for k in range(D // 32):
                    w = ebuf[b2][r, pl.ds(k * 16, 16)]
                    elo = lax.bitcast_convert_type(w << 16, jnp.float32)
                    ehi = lax.bitcast_convert_type(
                        w & jnp.int32(-65536), jnp.float32)
                    sl0 = pl.ds(k * 16, 16)
                    sl1 = pl.ds(D // 2 + k * 16, 16)
                    mbuf[b5][r, sl0] = jnp.maximum(
                        elo + gbuf[b2][r, sl0], 0.0)
                    mbuf[b5][r, sl1] = jnp.maximum(
                        ehi + gbuf[b2][r, sl1], 0.0)

            pltpu.make_async_copy(dst_hbm.at[pl.ds(ebase, CH)], dbuf[d10],
                                  sem_di.at[d10]).wait()
            scat_desc(b5, d10).start(add=True)

            @pl.when(j + MB < NCHUNK)
            def _psi():
                issue_sidx(j + MB, b5)

            @pl.when(j + GB < NCHUNK)
            def _pg():
                sb = (b5 + GB) % MB
                pltpu.make_async_copy(src_hbm.at[pl.ds(ebase, CH)],
                                      sbuf[sb], sem_si.at[sb]).wait()
                gather_desc(sb, b2).start()
                pltpu.async_copy(e_src(j + GB), ebuf[b2], sem_e.at[b2])
        return carry

    lax.fori_loop(0, NCHUNK // UNROLL, _group, 0)
    for b in range(MB):
        pltpu.make_async_copy(mbuf[b], agg_sh.at[dbuf[0]],
                              sem_sc.at[b]).wait()
    plsc.subcore_barrier()
    pltpu.sync_copy(agg_sh.at[pl.ds(s * RPS, RPS)],
                    out_hbm.at[c, pl.ds(s * RPS, RPS)])


def _mp_layer(h, e32, src, dst, l):
    body = functools.partial(_mp_body, l)
    return pl.kernel(
        body,
        out_type=jax.ShapeDtypeStruct((NC, PADN, D), jnp.float32),
        mesh=plsc.VectorSubcoreMesh(core_axis_name="c", subcore_axis_name="s",
                                    num_cores=NC, num_subcores=NS),
        scratch_types=[
            [pltpu.VMEM((CH,), jnp.int32)] * MB,
            [pltpu.VMEM((CH,), jnp.int32)] * DB,
            [pltpu.VMEM((CH, D), jnp.float32)] * GB,
            [pltpu.VMEM((CH, D // 2), jnp.int32)] * GB,
            [pltpu.VMEM((CH, D), jnp.float32)] * MB,
            pltpu.VMEM_SHARED((PADN, D), jnp.float32),
            pltpu.SemaphoreType.DMA((MB,)),
            pltpu.SemaphoreType.DMA((DB,)),
            pltpu.SemaphoreType.DMA((GB,)),
            pltpu.SemaphoreType.DMA((GB,)),
            pltpu.SemaphoreType.DMA((MB,)),
        ],
    )(h, e32, src, dst)


def _mlp_body(h_ref, a0_ref, a1_ref, w1_ref, b1_ref, w2_ref, b2_ref, out_ref):
    z = h_ref[...] + a0_ref[0] + a1_ref[0]
    u = jnp.maximum(jnp.dot(z, w1_ref[...], preferred_element_type=jnp.float32)
                    + b1_ref[...], 0.0)
    v = jnp.dot(u, w2_ref[...], preferred_element_type=jnp.float32) + b2_ref[...]
    out_ref[...] = jnp.maximum(v, 0.0)


def _mlp(h, agg2, W1l, b1l, W2l, b2l):
    grid = (N // BN,)
    return pl.pallas_call(
        _mlp_body,
        grid=grid,
        in_specs=[
            pl.BlockSpec((BN, D), lambda i: (i, 0)),
            pl.BlockSpec((1, BN, D), lambda i: (0, i, 0)),
            pl.BlockSpec((1, BN, D), lambda i: (1, i, 0)),
            pl.BlockSpec((D, D), lambda i: (0, 0)),
            pl.BlockSpec((1, D), lambda i: (0, 0)),
            pl.BlockSpec((D, D), lambda i: (0, 0)),
            pl.BlockSpec((1, D), lambda i: (0, 0)),
        ],
        out_specs=pl.BlockSpec((BN, D), lambda i: (i, 0)),
        out_shape=jax.ShapeDtypeStruct((N, D), jnp.float32),
    )(h, agg2, agg2, W1l, b1l, W2l, b2l)


def _pool_head_body(h_ref, batch_ref, wf1_ref, bf1_ref, wf2_ref, bf2_ref,
                    out_ref, acc_ref, cnt_ref):
    i = pl.program_id(0)

    @pl.when(i == 0)
    def _init():
        acc_ref[...] = jnp.zeros_like(acc_ref)
        cnt_ref[...] = jnp.zeros_like(cnt_ref)

    seg = batch_ref[0, 0]
    gids = jax.lax.broadcasted_iota(jnp.int32, (G, BP), 0)
    onehot = (gids == seg[None, :]).astype(jnp.float32)
    acc_ref[...] += jnp.dot(onehot, h_ref[...],
                            preferred_element_type=jnp.float32)
    cnt_ref[...] += jnp.sum(onehot, axis=1, keepdims=True)

    @pl.when(i == pl.num_programs(0) - 1)
    def _fin():
        pooled = acc_ref[...] / jnp.maximum(cnt_ref[...], 1.0)
        hid = jnp.maximum(
            jnp.dot(pooled, wf1_ref[...], preferred_element_type=jnp.float32)
            + bf1_ref[...], 0.0)
        out_ref[...] = (jnp.dot(hid, wf2_ref[...],
                                preferred_element_type=jnp.float32)
                        + bf2_ref[...])


def _pool_head(h, batch, Wf1, bf1, Wf2, bf2):
    batch3 = batch.reshape(N // BP, 1, BP)
    grid = (N // BP,)
    return pl.pallas_call(
        _pool_head_body,
        grid=grid,
        in_specs=[
            pl.BlockSpec((BP, D), lambda i: (i, 0)),
            pl.BlockSpec((1, 1, BP), lambda i: (i, 0, 0)),
            pl.BlockSpec((D, D), lambda i: (0, 0)),
            pl.BlockSpec((1, D), lambda i: (0, 0)),
            pl.BlockSpec((D, OUT), lambda i: (0, 0)),
            pl.BlockSpec((1, OUT), lambda i: (0, 0)),
        ],
        out_specs=pl.BlockSpec((G, OUT), lambda i: (0, 0)),
        out_shape=jax.ShapeDtypeStruct((G, OUT), jnp.float32),
        scratch_shapes=[
            pltpu.VMEM((G, D), jnp.float32),
            pltpu.VMEM((G, 1), jnp.float32),
        ],
    )(h, batch3, Wf1, bf1, Wf2, bf2)


def kernel(x, edge_index, edge_attr, batch, We, be, W1, b1, W2, b2,
           Wf1, bf1, Wf2, bf2):
    src = edge_index[0]
    dst = edge_index[1]
    e32 = _edge_embed(edge_attr, We, be.reshape(L, 1, D))
    h = x
    for l in range(L):
        agg2 = _mp_layer(h, e32, src, dst, l)
        h = _mlp(h, agg2, W1[l], b1[l].reshape(1, D),
                 W2[l], b2[l].reshape(1, D))
    return _pool_head(h, batch, Wf1, bf1.reshape(1, D), Wf2, bf2.reshape(1, OUT))


# speedup vs baseline: 3.6928x; 1.1442x over previous
import functools

import jax
import jax.numpy as jnp
import numpy as np
from jax import lax
from jax.experimental import pallas as pl
from jax.experimental.pallas import tpu as pltpu
from jax.experimental.pallas import tpu_sc as plsc


def _bf16_bits_rne(x):
    u = lax.bitcast_convert_type(x, jnp.uint32)
    r = ((u >> 16) & 1) + jnp.uint32(0x7FFF)
    return (u + r) >> 16

N = 10000
E = 320000
D = 128
ED = 16
OUT = 16
G = 64
L = 3

BE = 4000
BN = 1000
BP = 1000


def _edge_embed_body(ea_ref, we_ref, be_ref, out_ref):
    t = (jnp.dot(ea_ref[...], we_ref[...], preferred_element_type=jnp.float32)
         + be_ref[...])
    packed = (_bf16_bits_rne(t[:, : D // 2])
              | (_bf16_bits_rne(t[:, D // 2:]) << 16))
    out_ref[...] = lax.bitcast_convert_type(packed, jnp.int32)


def _edge_embed(edge_attr, Wel, bel):
    grid = (E // BE,)
    return pl.pallas_call(
        _edge_embed_body,
        grid=grid,
        in_specs=[
            pl.BlockSpec((BE, ED), lambda i: (i, 0)),
            pl.BlockSpec((ED, D), lambda i: (0, 0)),
            pl.BlockSpec((1, D), lambda i: (0, 0)),
        ],
        out_specs=pl.BlockSpec((BE, D // 2), lambda i: (i, 0)),
        out_shape=jax.ShapeDtypeStruct((E, D // 2), jnp.int32),
    )(edge_attr, Wel, bel)


NC = 2
NS = 16
NW = NC * NS
EPW = E // NW
CH = 40
NCHUNK = EPW // CH
PADN = 10112
RPS = PADN // NS
MB = 5
GB = 2
DB = 10
UNROLL = 10


def _mp_body(h_hbm, e_hbm, src_hbm, dst_hbm, out_hbm,
             sbuf, dbuf, gbuf, ebuf, mbuf, agg_sh,
             sem_si, sem_di, sem_g, sem_e, sem_sc):
    c = lax.axis_index("c")
    s = lax.axis_index("s")
    wid = c * NS + s
    ebase = wid * EPW

    def e_src(j):
        return e_hbm.at[pl.ds(ebase + j * CH, CH)]

    def issue_sidx(j, b):
        return pltpu.async_copy(src_hbm.at[pl.ds(ebase + j * CH, CH)],
                                sbuf[b], sem_si.at[b])

    def issue_didx(j, b):
        return pltpu.async_copy(dst_hbm.at[pl.ds(ebase + j * CH, CH)],
                                dbuf[b], sem_di.at[b])

    def gather_desc(sb, gb):
        return pltpu.make_async_copy(h_hbm.at[sbuf[sb]], gbuf[gb],
                                     sem_g.at[gb])

    def scat_desc(mb, db):
        return pltpu.make_async_copy(mbuf[mb], agg_sh.at[dbuf[db]],
                                     sem_sc.at[mb])

    @plsc.parallel_loop(0, CH, unroll=4)
    def _zrow(r):
        for k in range(D // 16):
            mbuf[0][r, pl.ds(k * 16, 16)] = jnp.zeros((16,), jnp.float32)

    for i in range(RPS // CH):
        pltpu.sync_copy(mbuf[0], agg_sh.at[pl.ds(s * RPS + i * CH, CH)])
    rem = RPS % CH
    if rem:
        pltpu.sync_copy(mbuf[0].at[pl.ds(0, rem)],
                        agg_sh.at[pl.ds(s * RPS + RPS - rem, rem)])
    plsc.subcore_barrier()

    for b in range(MB):
        issue_sidx(b, b)
    for b in range(DB):
        issue_didx(b, b)
    for b in range(GB):
        pltpu.async_copy(e_src(b), ebuf[b], sem_e.at[b])
        pltpu.make_async_copy(src_hbm.at[pl.ds(ebase, CH)], sbuf[b],
                              sem_si.at[b]).wait()
        gather_desc(b, b).start()

    def _group(g, carry):
        for u in range(UNROLL):
            b5 = u % MB
            b2 = u % GB
            d10 = u % DB
            j = g * UNROLL + u
            pltpu.make_async_copy(h_hbm.at[sbuf[b5]], gbuf[b2],
                                  sem_g.at[b2]).wait()
            pltpu.make_async_copy(e_src(j), ebuf[b2], sem_e.at[b2]).wait()

            @pl.when(j >= MB)
            def _drain():
                scat_desc(b5, d10).wait()

            @pl.when(jnp.logical_and(j >= MB, j + MB < NCHUNK))
            def _pdi():
                issue_didx(j + MB, (d10 + MB) % DB)

            @plsc.parallel_loop(0, CH, unroll=2)
            def _row(r):
                for k in range(D // 32):
                    w = ebuf[b2][r, pl.ds(k * 16, 16)]
                    elo = lax.bitcast_convert_type(w << 16, jnp.float32)
                    ehi = lax.bitcast_convert_type(
                        w & jnp.int32(-65536), jnp.float32)
                    sl0 = pl.ds(k * 16, 16)
                    sl1 = pl.ds(D // 2 + k * 16, 16)
                    mbuf[b5][r, sl0] = jnp.maximum(
                        elo + gbuf[b2][r, sl0], 0.0)
                    mbuf[b5][r, sl1] = jnp.maximum(
                        ehi + gbuf[b2][r, sl1], 0.0)

            pltpu.make_async_copy(dst_hbm.at[pl.ds(ebase, CH)], dbuf[d10],
                                  sem_di.at[d10]).wait()
            scat_desc(b5, d10).start(add=True)

            @pl.when(j + MB < NCHUNK)
            def _psi():
                issue_sidx(j + MB, b5)

            @pl.when(j + GB < NCHUNK)
            def _pg():
                sb = (b5 + GB) % MB
                pltpu.make_async_copy(src_hbm.at[pl.ds(ebase, CH)],
                                      sbuf[sb], sem_si.at[sb]).wait()
                gather_desc(sb, b2).start()
                pltpu.async_copy(e_src(j + GB), ebuf[b2], sem_e.at[b2])
        return carry

    lax.fori_loop(0, NCHUNK // UNROLL, _group, 0)
    for b in range(MB):
        pltpu.make_async_copy(mbuf[b], agg_sh.at[dbuf[0]],
                              sem_sc.at[b]).wait()
    plsc.subcore_barrier()
    pltpu.sync_copy(agg_sh.at[pl.ds(s * RPS, RPS)],
                    out_hbm.at[c, pl.ds(s * RPS, RPS)])


def _mp_layer(h, e32, src, dst):
    return pl.kernel(
        _mp_body,
        out_type=jax.ShapeDtypeStruct((NC, PADN, D), jnp.float32),
        mesh=plsc.VectorSubcoreMesh(core_axis_name="c", subcore_axis_name="s",
                                    num_cores=NC, num_subcores=NS),
        scratch_types=[
            [pltpu.VMEM((CH,), jnp.int32)] * MB,
            [pltpu.VMEM((CH,), jnp.int32)] * DB,
            [pltpu.VMEM((CH, D), jnp.float32)] * GB,
            [pltpu.VMEM((CH, D // 2), jnp.int32)] * GB,
            [pltpu.VMEM((CH, D), jnp.float32)] * MB,
            pltpu.VMEM_SHARED((PADN, D), jnp.float32),
            pltpu.SemaphoreType.DMA((MB,)),
            pltpu.SemaphoreType.DMA((DB,)),
            pltpu.SemaphoreType.DMA((GB,)),
            pltpu.SemaphoreType.DMA((GB,)),
            pltpu.SemaphoreType.DMA((MB,)),
        ],
    )(h, e32, src, dst)


def _mlp_body(h_ref, a0_ref, a1_ref, w1_ref, b1_ref, w2_ref, b2_ref, out_ref):
    z = h_ref[...] + a0_ref[0] + a1_ref[0]
    u = jnp.maximum(jnp.dot(z, w1_ref[...], preferred_element_type=jnp.float32)
                    + b1_ref[...], 0.0)
    v = jnp.dot(u, w2_ref[...], preferred_element_type=jnp.float32) + b2_ref[...]
    out_ref[...] = jnp.maximum(v, 0.0)


def _mlp(h, agg2, W1l, b1l, W2l, b2l):
    grid = (N // BN,)
    return pl.pallas_call(
        _mlp_body,
        grid=grid,
        in_specs=[
            pl.BlockSpec((BN, D), lambda i: (i, 0)),
            pl.BlockSpec((1, BN, D), lambda i: (0, i, 0)),
            pl.BlockSpec((1, BN, D), lambda i: (1, i, 0)),
            pl.BlockSpec((D, D), lambda i: (0, 0)),
            pl.BlockSpec((1, D), lambda i: (0, 0)),
            pl.BlockSpec((D, D), lambda i: (0, 0)),
            pl.BlockSpec((1, D), lambda i: (0, 0)),
        ],
        out_specs=pl.BlockSpec((BN, D), lambda i: (i, 0)),
        out_shape=jax.ShapeDtypeStruct((N, D), jnp.float32),
    )(h, agg2, agg2, W1l, b1l, W2l, b2l)


def _pool_head_body(h_ref, batch_ref, wf1_ref, bf1_ref, wf2_ref, bf2_ref,
                    out_ref, acc_ref, cnt_ref):
    i = pl.program_id(0)

    @pl.when(i == 0)
    def _init():
        acc_ref[...] = jnp.zeros_like(acc_ref)
        cnt_ref[...] = jnp.zeros_like(cnt_ref)

    seg = batch_ref[0, 0]
    gids = jax.lax.broadcasted_iota(jnp.int32, (G, BP), 0)
    onehot = (gids == seg[None, :]).astype(jnp.float32)
    acc_ref[...] += jnp.dot(onehot, h_ref[...],
                            preferred_element_type=jnp.float32)
    cnt_ref[...] += jnp.sum(onehot, axis=1, keepdims=True)

    @pl.when(i == pl.num_programs(0) - 1)
    def _fin():
        pooled = acc_ref[...] / jnp.maximum(cnt_ref[...], 1.0)
        hid = jnp.maximum(
            jnp.dot(pooled, wf1_ref[...], preferred_element_type=jnp.float32)
            + bf1_ref[...], 0.0)
        out_ref[...] = (jnp.dot(hid, wf2_ref[...],
                                preferred_element_type=jnp.float32)
                        + bf2_ref[...])


def _pool_head(h, batch, Wf1, bf1, Wf2, bf2):
    batch3 = batch.reshape(N // BP, 1, BP)
    grid = (N // BP,)
    return pl.pallas_call(
        _pool_head_body,
        grid=grid,
        in_specs=[
            pl.BlockSpec((BP, D), lambda i: (i, 0)),
            pl.BlockSpec((1, 1, BP), lambda i: (i, 0, 0)),
            pl.BlockSpec((D, D), lambda i: (0, 0)),
            pl.BlockSpec((1, D), lambda i: (0, 0)),
            pl.BlockSpec((D, OUT), lambda i: (0, 0)),
            pl.BlockSpec((1, OUT), lambda i: (0, 0)),
        ],
        out_specs=pl.BlockSpec((G, OUT), lambda i: (0, 0)),
        out_shape=jax.ShapeDtypeStruct((G, OUT), jnp.float32),
        scratch_shapes=[
            pltpu.VMEM((G, D), jnp.float32),
            pltpu.VMEM((G, 1), jnp.float32),
        ],
    )(h, batch3, Wf1, bf1, Wf2, bf2)


def kernel(x, edge_index, edge_attr, batch, We, be, W1, b1, W2, b2,
           Wf1, bf1, Wf2, bf2):
    src = edge_index[0]
    dst = edge_index[1]
    h = x
    for l in range(L):
        e32 = _edge_embed(edge_attr, We[l], be[l].reshape(1, D))
        agg2 = _mp_layer(h, e32, src, dst)
        h = _mlp(h, agg2, W1[l], b1[l].reshape(1, D),
                 W2[l], b2[l].reshape(1, D))
    return _pool_head(h, batch, Wf1, bf1.reshape(1, D), Wf2, bf2.reshape(1, OUT))
